# baseline (speedup 1.0000x reference)
import functools

import jax
import jax.numpy as jnp
from jax import lax
from jax.experimental import pallas as pl
from jax.experimental.pallas import tpu as pltpu
from jax.experimental.pallas import tpu_sc as plsc

N_ATOMS = 10000
DRUG_DIM = 128
OUT = 128
HEADS = 4
DH = 32
N_DRUG = 500
N_CLINE = 200
CLINE_DIM = 512
N_GRAPH = 700
N_SYN = 20000
B = 20000
EPS = 1e-5


def _bn(x, g, b):
    return x / jnp.sqrt(1.0 + EPS) * g + b


_BPAD = 20224


def _sc_gather(table, idx):
    info = plsc.get_sparse_core_info()
    nw = info.num_cores * info.num_subcores
    bpw = _BPAD // nw
    nc = info.num_cores
    mesh = plsc.VectorSubcoreMesh(core_axis_name="c", subcore_axis_name="s")

    @functools.partial(
        pl.kernel, mesh=mesh,
        out_type=jax.ShapeDtypeStruct((_BPAD, OUT), jnp.float32),
        scratch_types=[
            pltpu.VMEM((bpw,), jnp.int32),
            pltpu.VMEM((bpw, OUT), jnp.float32),
            pltpu.SemaphoreType.DMA,
        ],
    )
    def k(table_hbm, idx_hbm, out_hbm, idx_v, rows_v, sem):
        wid = lax.axis_index("s") * nc + lax.axis_index("c")
        base = wid * bpw
        pltpu.sync_copy(idx_hbm.at[pl.ds(base, bpw)], idx_v)
        pltpu.async_copy(table_hbm.at[idx_v], rows_v, sem).wait()
        pltpu.sync_copy(rows_v, out_hbm.at[pl.ds(base, bpw)])

    return k(table, idx)


def _triplet_gather(ge, ia, ib, ic):
    pad = jnp.zeros((_BPAD - B,), jnp.int32)
    outs = [_sc_gather(ge, jnp.concatenate([i.astype(jnp.int32), pad]))[:B]
            for i in (ia, ib, ic)]
    return jnp.concatenate(outs, axis=-1)


def _dec_kernel(cand_ref, w1t, b1, w2t, b2, w3, b3, out_ref):
    h = jnp.maximum(cand_ref[:] @ w1t[:] + b1[:], 0.0)
    h = jnp.maximum(h @ w2t[:] + b2[:], 0.0)
    logits = jnp.sum(h * w3[:], axis=1, keepdims=True) + b3[0, 0]
    out_ref[:] = jax.nn.sigmoid(logits)


def _dec_mlp(cand, d):
    blk = 1000
    grid = B // blk
    w1t = d["W1"].T
    w2t = d["W2"].T
    w3 = d["W3"]
    b1 = d["b1"][None, :]
    b2 = d["b2"][None, :]
    b3 = d["b3"][None, :]
    out = pl.pallas_call(
        _dec_kernel,
        grid=(grid,),
        in_specs=[
            pl.BlockSpec((blk, 384), lambda i: (i, 0)),
            pl.BlockSpec((384, 192), lambda i: (0, 0)),
            pl.BlockSpec((1, 192), lambda i: (0, 0)),
            pl.BlockSpec((192, 96), lambda i: (0, 0)),
            pl.BlockSpec((1, 96), lambda i: (0, 0)),
            pl.BlockSpec((1, 96), lambda i: (0, 0)),
            pl.BlockSpec((1, 1), lambda i: (0, 0)),
        ],
        out_specs=pl.BlockSpec((blk, 1), lambda i: (i, 0)),
        out_shape=jax.ShapeDtypeStruct((B, 1), jnp.float32),
    )(cand, w1t, b1, w2t, b2, w3, b3)
    return out[:, 0]


def _tconv(x, ei, pp, n):
    src, dst = ei[0], ei[1]
    q = x @ pp["Wq"].T + pp["bq"]
    k = x @ pp["Wk"].T + pp["bk"]
    v = x @ pp["Wv"].T + pp["bv"]
    qk = q[dst] * k[src]
    a = jnp.sum(qk.reshape(-1, HEADS, DH), axis=-1) / jnp.sqrt(float(DH))
    ae = jnp.exp(a)
    den = jax.ops.segment_sum(ae, dst, num_segments=n)
    w = jnp.repeat(ae, DH, axis=1)
    num = jax.ops.segment_sum(v[src] * w, dst, num_segments=n)
    out = num / (jnp.repeat(den, DH, axis=1) + 1e-16)
    return out


def _tblock(x, ei, pp, n):
    h = jax.nn.relu(_tconv(x, ei, pp, n))
    return _bn(h, pp["bn_g"], pp["bn_b"])


def _hgconv(X, H, w, Theta, bias, n, m):
    ni, ei = H[0], H[1]
    Xt = X @ Theta.T
    Bdeg = jax.ops.segment_sum(jnp.ones((ni.shape[0],), X.dtype), ei, num_segments=m)
    Binv = jnp.where(Bdeg > 0, 1.0 / Bdeg, 0.0)
    D = jax.ops.segment_sum(w[ei], ni, num_segments=n)
    Dinv = jnp.where(D > 0, 1.0 / D, 0.0)
    ef = jax.ops.segment_sum(Xt[ni], ei, num_segments=m) * Binv[:, None]
    out = jax.ops.segment_sum(w[ei][:, None] * ef[ei], ni, num_segments=n) * Dinv[:, None]
    return out + bias


def kernel(drug_x, cline_x, hyperedge_weight, params, drug_adj, ibatch, H,
           druga_id, drugb_id, cline_id):
    p = params
    x = _tblock(drug_x, drug_adj, p["drug_first"], N_ATOMS)
    for pp in p["drug_same"]:
        x = x + _tblock(x, drug_adj, pp, N_ATOMS)
    cnt = jax.ops.segment_sum(jnp.ones((N_ATOMS,), x.dtype), ibatch, num_segments=N_DRUG)
    drug_emb = jax.ops.segment_sum(x, ibatch, num_segments=N_DRUG) / jnp.maximum(cnt, 1.0)[:, None]

    c = jnp.tanh(cline_x @ p["cline_first"]["W"].T + p["cline_first"]["b"])
    for pp in p["cline_same"]:
        c = c + jax.nn.relu(c @ pp["W"].T + pp["b"])

    X = jnp.concatenate([drug_emb, c], axis=0)
    identity = X
    for pp in p["ref"]:
        h = _bn(X, pp["bn_g"], pp["bn_b"])
        h = jax.nn.relu(_hgconv(h, H, hyperedge_weight, pp["Theta"], pp["hb"], N_GRAPH, N_SYN))
        gate = jax.nn.sigmoid(X @ pp["wW"].T + pp["wb"])
        X = X + h * gate
    graph_embed = (X + identity) + X

    cand = _triplet_gather(graph_embed, druga_id, drugb_id, cline_id)
    return _dec_mlp(cand, p["dec"])
